# Initial kernel scaffold; baseline (speedup 1.0000x reference)
import jax
import jax.numpy as jnp
from jax.experimental import pallas as pl


def kernel(pos, ef, edge_src, edge_dst, ag_order, continuing_ag, joint_action_prev, W1, b1, W2, b2, Wm, bm, Wu, bu, Wb1, bb1, Wb2, bb2):
    raise NotImplementedError("write your pallas kernel here")


# speedup vs baseline: 1.1695x; 1.1695x over previous
import functools

import jax
import jax.numpy as jnp
from jax import lax
from jax.experimental import pallas as pl
from jax.experimental.pallas import tpu as pltpu

N_AG = 64
N_TASK = 8192
N_NODES = N_AG + N_TASK
E = 262144
D = 64


def _lrelu(x):
    return jnp.where(x >= 0, x, 0.01 * x)


def _sample_body(ao_ref, cont_ref, jap_ref, pol_ref, g_ref, out_ref, tk_ref, act_ref):
    tk_ref[...] = jnp.zeros_like(tk_ref)
    act_ref[...] = jnp.zeros_like(act_ref)
    iota = lax.broadcasted_iota(jnp.int32, (1, N_TASK), 1)

    def body(itr, _):
        agent = ao_ref[itr]
        taken = tk_ref[...]
        row = jnp.where(taken != 0, 0.0, pol_ref[pl.ds(agent, 1), :])
        allzero = jnp.max(row) <= 0.0
        sc = jnp.log(jnp.clip(row, 1e-20, None)) + g_ref[pl.ds(itr, 1), :]
        m2 = jnp.max(sc)
        samp = jnp.min(jnp.where(sc == m2, iota, jnp.int32(2**31 - 1)))
        action = jnp.where(cont_ref[agent] != 0, jap_ref[agent], samp)
        action = jnp.where(allzero, jnp.int32(-1), action).astype(jnp.int32)
        lane = lax.broadcasted_iota(jnp.int32, (1, N_AG), 1)
        act_ref[...] = jnp.where(lane == itr, action, act_ref[...])
        col = jnp.maximum(action, 0)
        tk_ref[...] = jnp.where((iota == col) & jnp.logical_not(allzero),
                                jnp.int32(1), taken)
        return 0

    lax.fori_loop(0, N_AG, body, 0)
    out_ref[...] = act_ref[...]


@jax.jit
def _sample(policy, G, ag_order, cont_i32, jap):
    out = pl.pallas_call(
        _sample_body,
        out_shape=jax.ShapeDtypeStruct((1, N_AG), jnp.int32),
        in_specs=[
            pl.BlockSpec(memory_space=pltpu.SMEM),
            pl.BlockSpec(memory_space=pltpu.SMEM),
            pl.BlockSpec(memory_space=pltpu.SMEM),
            pl.BlockSpec(memory_space=pltpu.VMEM),
            pl.BlockSpec(memory_space=pltpu.VMEM),
        ],
        out_specs=pl.BlockSpec(memory_space=pltpu.VMEM),
        scratch_shapes=[
            pltpu.VMEM((1, N_TASK), jnp.int32),
            pltpu.VMEM((1, N_AG), jnp.int32),
        ],
    )(ag_order, cont_i32, jap, policy, G)
    return out[0]


def _gumbel_table():
    base = jax.random.key(42)
    subs = jax.vmap(lambda i: jax.random.fold_in(base, i))(jnp.arange(N_AG))
    return jax.vmap(lambda k: jax.random.gumbel(k, (N_TASK,), jnp.float32))(subs)


def kernel(pos, ef, edge_src, edge_dst, ag_order, continuing_ag, joint_action_prev,
           W1, b1, W2, b2, Wm, bm, Wu, bu, Wb1, bb1, Wb2, bb2):
    nf = _lrelu(_lrelu(pos @ W1 + b1) @ W2 + b2)
    src = edge_src
    dst = edge_dst + N_AG
    h_s = nf[src]
    h_d = nf[dst]
    m_fwd = _lrelu(jnp.concatenate([h_s, h_d, ef], axis=-1) @ Wm + bm)
    m_bwd = _lrelu(jnp.concatenate([h_d, h_s, ef], axis=-1) @ Wm + bm)
    agg = jnp.zeros((N_NODES, D), jnp.float32).at[dst].add(m_fwd).at[src].add(m_bwd)
    h = nf + _lrelu(jnp.concatenate([nf, agg], axis=-1) @ Wu + bu)
    scores = (_lrelu(jnp.concatenate([h[src], h[dst]], axis=-1) @ Wb1 + bb1) @ Wb2 + bb2)[:, 0]
    logits = jnp.full((N_AG, N_TASK), -1e9, jnp.float32).at[edge_src, edge_dst].max(scores)
    policy = jax.nn.softmax(logits, axis=-1)

    G = _gumbel_table()
    return _sample(policy, G, ag_order, continuing_ag.astype(jnp.int32),
                   joint_action_prev.astype(jnp.int32))


# speedup vs baseline: 2.8870x; 2.4687x over previous
import jax
import jax.numpy as jnp
from jax import lax
from jax.experimental import pallas as pl
from jax.experimental.pallas import tpu as pltpu
from jax.experimental.pallas import tpu_sc as plsc

N_AG = 64
N_TASK = 8192
N_NODES = N_AG + N_TASK
E = 262144
D = 64

NC = 2
NS = 16
NW = NC * NS
EPT = E // NW
BB = 128
N_PAD = 8320
RPT = N_PAD // NS
CH = 2048
CAP = EPT + 1024


def _lrelu(x):
    return jnp.where(x >= 0, x, 0.01 * x)


def _k1_body(pos_ref, w1_ref, b1_ref, w2_ref, b2_ref, wms_ref, wmd_ref,
             nf_ref, ab_ref):
    x = jnp.dot(pos_ref[...], w1_ref[...], preferred_element_type=jnp.float32)
    x = _lrelu(x + b1_ref[...])
    nf = _lrelu(jnp.dot(x, w2_ref[...], preferred_element_type=jnp.float32)
                + b2_ref[...])
    nf_ref[...] = nf
    ab_ref[:, :D] = jnp.dot(nf, wms_ref[...], preferred_element_type=jnp.float32)
    ab_ref[:, D:] = jnp.dot(nf, wmd_ref[...], preferred_element_type=jnp.float32)


@jax.jit
def _k1(pos, W1, b1, W2, b2, Wms, Wmd):
    return pl.pallas_call(
        _k1_body,
        out_shape=(jax.ShapeDtypeStruct((N_NODES, D), jnp.float32),
                   jax.ShapeDtypeStruct((N_NODES, 2 * D), jnp.float32)),
    )(pos, W1, b1.reshape(1, D), W2, b2.reshape(1, D), Wms, Wmd)


def _k2_body(ab_hbm, aag_hbm, bag_hbm, wme_hbm, bm_hbm, src_hbm, d64_hbm,
             ef0_hbm, ef1_hbm, ef2_hbm, mf_hbm, bwd_hbm,
             aag_v, bag_v, wme_v, bm_v, src_v, didx_v,
             ef0_v, ef1_v, ef2_v, rows_v, mfwd_v, mbwd_v, aggag_v, sem):
    c = lax.axis_index("c")
    s = lax.axis_index("s")
    wid = c * NS + s

    zrow = jnp.zeros((16,), jnp.float32)

    def zb(r, _):
        for k in range(4):
            aggag_v[r, pl.ds(16 * k, 16)] = zrow
        return 0
    lax.fori_loop(0, N_AG, zb, 0)

    pltpu.sync_copy(aag_hbm, aag_v)
    pltpu.sync_copy(bag_hbm, bag_v)
    pltpu.sync_copy(wme_hbm, wme_v)
    pltpu.sync_copy(bm_hbm, bm_v)

    w0 = [wme_v[0, pl.ds(16 * k, 16)] for k in range(4)]
    w1 = [wme_v[1, pl.ds(16 * k, 16)] for k in range(4)]
    w2 = [wme_v[2, pl.ds(16 * k, 16)] for k in range(4)]
    bmv = [bm_v[pl.ds(16 * k, 16)] for k in range(4)]

    def batch(b, _):
        eb = wid * EPT + b * BB
        bidx = wid * (EPT // BB) + b
        pltpu.sync_copy(src_hbm.at[bidx], src_v)
        pltpu.sync_copy(d64_hbm.at[bidx], didx_v)
        pltpu.sync_copy(ef0_hbm.at[pl.ds(eb, BB)], ef0_v)
        pltpu.sync_copy(ef1_hbm.at[pl.ds(eb, BB)], ef1_v)
        pltpu.sync_copy(ef2_hbm.at[pl.ds(eb, BB)], ef2_v)
        pltpu.async_copy(ab_hbm.at[didx_v.at[0]], rows_v, sem).wait()

        def grp(g, _):
            src16 = src_v[0, pl.ds(g * 16, 16)]
            e0_16 = ef0_v[pl.ds(g * 16, 16)]
            e1_16 = ef1_v[pl.ds(g * 16, 16)]
            e2_16 = ef2_v[pl.ds(g * 16, 16)]
            for lane in range(16):
                i = g * 16 + lane
                si = src16[lane]
                e0 = e0_16[lane]
                e1 = e1_16[lane]
                e2 = e2_16[lane]
                for k in range(4):
                    sl = pl.ds(16 * k, 16)
                    cvec = w0[k] * e0 + w1[k] * e1 + w2[k] * e2 + bmv[k]
                    ad = rows_v[i, sl]
                    bd = rows_v[i, pl.ds(D + 16 * k, 16)]
                    asv = aag_v[si, sl]
                    bsv = bag_v[si, sl]
                    mfwd_v[i, sl] = _lrelu(asv + bd + cvec)
                    aggag_v[si, sl] = aggag_v[si, sl] + _lrelu(ad + bsv + cvec)
            return 0
        lax.fori_loop(0, BB // 16, grp, 0)
        pltpu.sync_copy(mfwd_v, mf_hbm.at[pl.ds(eb, BB)])
        return 0
    lax.fori_loop(0, EPT // BB, batch, 0)

    pltpu.sync_copy(aggag_v, bwd_hbm.at[pl.ds(wid * N_AG, N_AG)])


@jax.jit
def _k2(ab, aag, bag, wme, bm, src, d64, ef0, ef1, ef2):
    mesh = plsc.VectorSubcoreMesh(core_axis_name="c", subcore_axis_name="s")
    f = pl.kernel(
        _k2_body,
        out_type=(jax.ShapeDtypeStruct((E, D), jnp.float32),
                  jax.ShapeDtypeStruct((NW * N_AG, D), jnp.float32)),
        mesh=mesh,
        scratch_types=[
            pltpu.VMEM((N_AG, D), jnp.float32),
            pltpu.VMEM((N_AG, D), jnp.float32),
            pltpu.VMEM((3, D), jnp.float32),
            pltpu.VMEM((D,), jnp.float32),
            pltpu.VMEM((1, BB), jnp.int32),
            pltpu.VMEM((1, BB), jnp.int32),
            pltpu.VMEM((BB,), jnp.float32),
            pltpu.VMEM((BB,), jnp.float32),
            pltpu.VMEM((BB,), jnp.float32),
            pltpu.VMEM((BB, 2 * D), jnp.float32),
            pltpu.VMEM((BB, D), jnp.float32),
            pltpu.VMEM((BB, D), jnp.float32),
            pltpu.VMEM((N_AG, D), jnp.float32),
            pltpu.SemaphoreType.DMA,
        ],
    )
    return f(ab, aag, bag, wme, bm,
             src.reshape(E // BB, 1, BB), d64.reshape(E // BB, 1, BB),
             ef0, ef1, ef2)


def _k3_body(aggp_ref, nf_ref, wu1_ref, wu2_ref, bu_ref, wb1s_ref, wb1d_ref,
             bb1_ref, pag_ref, q_ref):
    agg = aggp_ref[...]
    nf = nf_ref[...]
    u = (jnp.dot(nf, wu1_ref[...], preferred_element_type=jnp.float32)
         + jnp.dot(agg, wu2_ref[...], preferred_element_type=jnp.float32)
         + bu_ref[...])
    h = nf + _lrelu(u)
    q_ref[:, :D] = jnp.dot(h, wb1d_ref[...], preferred_element_type=jnp.float32)
    q_ref[:, D:] = jnp.zeros((N_NODES, D), jnp.float32)
    pag_ref[...] = (jnp.dot(h[:N_AG], wb1s_ref[...],
                            preferred_element_type=jnp.float32) + bb1_ref[...])


@jax.jit
def _k3(aggp, nf, Wu1, Wu2, bu, Wb1s, Wb1d, bb1):
    return pl.pallas_call(
        _k3_body,
        out_shape=(jax.ShapeDtypeStruct((N_AG, D), jnp.float32),
                   jax.ShapeDtypeStruct((N_NODES, 2 * D), jnp.float32)),
    )(aggp, nf, Wu1, Wu2, bu.reshape(1, D), Wb1s, Wb1d, bb1.reshape(1, D))


def _hsum16(acc, iota):
    for sh in (8, 4, 2, 1):
        p = (iota + sh) & 15
        acc = acc + acc.at[p].get(mode="promise_in_bounds")
    return acc[0]


def _bf16r(x):
    c = x * jnp.float32(65537.0)
    return c - (c - x)


def _k4_body(q_hbm, pag_hbm, wb2_hbm, src_hbm, d64_hbm, s_hbm,
             pag_v, wb2_v, src_v, d64_v, qrows_v, accs_v, sem):
    c = lax.axis_index("c")
    s = lax.axis_index("s")
    wid = c * NS + s

    pltpu.sync_copy(pag_hbm, pag_v)
    pltpu.sync_copy(wb2_hbm, wb2_v)
    wvecs = [wb2_v[pl.ds(16 * k, 16)] for k in range(4)]

    def batch(b, _):
        eb = wid * EPT + b * BB
        bidx = wid * (EPT // BB) + b
        pltpu.sync_copy(src_hbm.at[bidx], src_v)
        pltpu.sync_copy(d64_hbm.at[bidx], d64_v)
        pltpu.async_copy(q_hbm.at[d64_v.at[0]], qrows_v, sem).wait()

        def grp(g, _):
            src16 = src_v[0, pl.ds(g * 16, 16)]
            for lane in range(16):
                i = g * 16 + lane
                si = src16[lane]
                acc = jnp.zeros((16,), jnp.float32)
                for k in range(4):
                    sl = pl.ds(16 * k, 16)
                    t = _bf16r(_lrelu(pag_v[si, sl] + qrows_v[i, sl]))
                    acc = acc + t * wvecs[k]
                accs_v[i, pl.ds(0, 16)] = acc
            return 0
        lax.fori_loop(0, BB // 16, grp, 0)
        pltpu.sync_copy(accs_v, s_hbm.at[pl.ds(eb, BB)])
        return 0
    lax.fori_loop(0, EPT // BB, batch, 0)


@jax.jit
def _k4(q, pag, wb2, src, d64):
    mesh = plsc.VectorSubcoreMesh(core_axis_name="c", subcore_axis_name="s")
    f = pl.kernel(
        _k4_body,
        out_type=jax.ShapeDtypeStruct((E, 16), jnp.float32),
        mesh=mesh,
        scratch_types=[
            pltpu.VMEM((N_AG, D), jnp.float32),
            pltpu.VMEM((D,), jnp.float32),
            pltpu.VMEM((1, BB), jnp.int32),
            pltpu.VMEM((1, BB), jnp.int32),
            pltpu.VMEM((BB, 2 * D), jnp.float32),
            pltpu.VMEM((BB, 16), jnp.float32),
            pltpu.SemaphoreType.DMA,
        ],
    )
    return f(q, pag, wb2,
             src.reshape(E // BB, 1, BB), d64.reshape(E // BB, 1, BB))


def _k5_body(src_hbm, dst_hbm, s_hbm, out_hbm,
             loc_v, dbuf_v, vbuf_v, schunk_v, dchunk_v, vchunk_v):
    c = lax.axis_index("c")
    s = lax.axis_index("s")
    wid = c * NS + s
    neg = jnp.full((16,), -1e9, jnp.float32)
    iota = lax.iota(jnp.int32, 16)

    def zr(j, _):
        loc_v[pl.ds(j * 16, 16)] = neg
        return 0
    lax.fori_loop(0, (2 * N_TASK + 32) // 16, zr, 0)

    def chunk(ch, o):
        pltpu.sync_copy(src_hbm.at[pl.ds(ch * CH, CH)], schunk_v)
        pltpu.sync_copy(dst_hbm.at[pl.ds(ch * CH, CH)], dchunk_v)
        pltpu.sync_copy(s_hbm.at[pl.ds(ch * CH, CH)], vchunk_v)

        def grp(g, o):
            s16 = schunk_v[pl.ds(g * 16, 16)]
            d16 = dchunk_v[pl.ds(g * 16, 16)]
            v16 = vchunk_v[pl.ds(g * 16, 16)]
            msk = lax.shift_right_logical(s16, 1) == wid
            flat = (s16 & 1) * N_TASK + d16
            osafe = jnp.minimum(o, CAP - 16)
            plsc.store_compressed(dbuf_v.at[pl.ds(osafe, 16)], flat, mask=msk)
            plsc.store_compressed(vbuf_v.at[pl.ds(osafe, 16)], v16, mask=msk)
            k = plsc.all_reduce_population_count(msk)[0]
            return jnp.minimum(o + k, CAP - 16)
        return lax.fori_loop(0, CH // 16, grp, o)
    o = lax.fori_loop(0, E // CH, chunk, 0)
    dbuf_v[pl.ds(o, 16)] = jnp.full((16,), 2 * N_TASK, jnp.int32)

    def drain(r, _):
        d16 = dbuf_v[pl.ds(r * 16, 16)]
        v16 = vbuf_v[pl.ds(r * 16, 16)]
        for lane in range(16):
            d = d16[lane]
            v = v16[lane]
            w = loc_v[pl.ds(d, 16)]
            loc_v[pl.ds(d, 16)] = jnp.where(iota == 0,
                                            jnp.maximum(w[0], v), w)
        return 0
    lax.fori_loop(0, lax.div(o + 15, 16), drain, 0)

    pltpu.sync_copy(loc_v.at[pl.ds(0, 2 * N_TASK)],
                    out_hbm.at[pl.ds(2 * wid * N_TASK, 2 * N_TASK)])


@jax.jit
def _k5(src, dst, svals):
    mesh = plsc.VectorSubcoreMesh(core_axis_name="c", subcore_axis_name="s")
    f = pl.kernel(
        _k5_body,
        out_type=jax.ShapeDtypeStruct((N_AG * N_TASK,), jnp.float32),
        mesh=mesh,
        scratch_types=[
            pltpu.VMEM((2 * N_TASK + 32,), jnp.float32),
            pltpu.VMEM((CAP,), jnp.int32),
            pltpu.VMEM((CAP,), jnp.float32),
            pltpu.VMEM((CH,), jnp.int32),
            pltpu.VMEM((CH,), jnp.int32),
            pltpu.VMEM((CH,), jnp.float32),
        ],
    )
    return f(src, dst, svals)


def _k6_body(ao_ref, cont_ref, jap_ref, lg_ref, g_ref, out_ref,
             pol_ref, tk_ref, act_ref):
    lg = lg_ref[...]
    ex = jnp.exp(lg - jnp.max(lg, axis=1, keepdims=True))
    pol_ref[...] = ex / jnp.sum(ex, axis=1, keepdims=True)
    tk_ref[...] = jnp.zeros_like(tk_ref)
    act_ref[...] = jnp.zeros_like(act_ref)
    iota = lax.broadcasted_iota(jnp.int32, (1, N_TASK), 1)

    def body(itr, _):
        agent = ao_ref[itr]
        taken = tk_ref[...]
        row = jnp.where(taken != 0, 0.0, pol_ref[pl.ds(agent, 1), :])
        allzero = jnp.max(row) <= 0.0
        sc = jnp.log(jnp.clip(row, 1e-20, None)) + g_ref[pl.ds(itr, 1), :]
        m2 = jnp.max(sc)
        samp = jnp.min(jnp.where(sc == m2, iota, jnp.int32(2**31 - 1)))
        action = jnp.where(cont_ref[agent] != 0, jap_ref[agent], samp)
        action = jnp.where(allzero, jnp.int32(-1), action).astype(jnp.int32)
        lane = lax.broadcasted_iota(jnp.int32, (1, N_AG), 1)
        act_ref[...] = jnp.where(lane == itr, action, act_ref[...])
        col = jnp.maximum(action, 0)
        tk_ref[...] = jnp.where((iota == col) & jnp.logical_not(allzero),
                                jnp.int32(1), taken)
        return 0

    lax.fori_loop(0, N_AG, body, 0)
    out_ref[...] = act_ref[...]


@jax.jit
def _k6(logits, G, ag_order, cont_i32, jap):
    out = pl.pallas_call(
        _k6_body,
        out_shape=jax.ShapeDtypeStruct((1, N_AG), jnp.int32),
        in_specs=[
            pl.BlockSpec(memory_space=pltpu.SMEM),
            pl.BlockSpec(memory_space=pltpu.SMEM),
            pl.BlockSpec(memory_space=pltpu.SMEM),
            pl.BlockSpec(memory_space=pltpu.VMEM),
            pl.BlockSpec(memory_space=pltpu.VMEM),
        ],
        out_specs=pl.BlockSpec(memory_space=pltpu.VMEM),
        scratch_shapes=[
            pltpu.VMEM((N_AG, N_TASK), jnp.float32),
            pltpu.VMEM((1, N_TASK), jnp.int32),
            pltpu.VMEM((1, N_AG), jnp.int32),
        ],
    )(ag_order, cont_i32, jap, logits, G)
    return out[0]


def _gumbel_table():
    base = jax.random.key(42)
    subs = jax.vmap(lambda i: jax.random.fold_in(base, i))(jnp.arange(N_AG))
    return jax.vmap(lambda k: jax.random.gumbel(k, (N_TASK,), jnp.float32))(subs)


def kernel(pos, ef, edge_src, edge_dst, ag_order, continuing_ag, joint_action_prev,
           W1, b1, W2, b2, Wm, bm, Wu, bu, Wb1, bb1, Wb2, bb2):
    d64 = edge_dst + N_AG
    nf, ab = _k1(pos, W1, b1, W2, b2, Wm[:D], Wm[D:2 * D])
    aag = ab[:N_AG, :D]
    bag = ab[:N_AG, D:]
    efb = ef.astype(jnp.bfloat16).astype(jnp.float32)
    wmeb = Wm[2 * D:].astype(jnp.bfloat16).astype(jnp.float32)
    mf, bwd = _k2(ab, aag, bag, wmeb, bm, edge_src, d64,
                  efb[:, 0], efb[:, 1], efb[:, 2])
    agg = jnp.zeros((N_NODES, D), jnp.float32).at[d64].add(mf)
    agg = agg.at[:N_AG].add(bwd.reshape(NW, N_AG, D).sum(0))
    pag, q = _k3(agg, nf, Wu[:D], Wu[D:], bu, Wb1[:D], Wb1[D:], bb1)
    wb2b = Wb2[:, 0].astype(jnp.bfloat16).astype(jnp.float32)
    parts = _k4(q, pag, wb2b, edge_src, d64)
    scores = parts.sum(axis=1) + bb2[0]
    logits = jnp.full((N_AG, N_TASK), -1e9, jnp.float32).at[
        edge_src, edge_dst].max(scores)
    G = _gumbel_table()
    return _k6(logits, G, ag_order, continuing_ag.astype(jnp.int32),
               joint_action_prev.astype(jnp.int32))


# speedup vs baseline: 2.8904x; 1.0012x over previous
import jax
import jax.numpy as jnp
from jax import lax
from jax.experimental import pallas as pl
from jax.experimental.pallas import tpu as pltpu
from jax.experimental.pallas import tpu_sc as plsc

N_AG = 64
N_TASK = 8192
N_NODES = N_AG + N_TASK
E = 262144
D = 64

NC = 2
NS = 16
NW = NC * NS
EPT = E // NW
BB = 128


def _lrelu(x):
    return jnp.where(x >= 0, x, 0.01 * x)


def _k1_body(pos_ref, w1_ref, b1_ref, w2_ref, b2_ref, wms_ref, wmd_ref,
             nf_ref, ab_ref):
    x = jnp.dot(pos_ref[...], w1_ref[...], preferred_element_type=jnp.float32)
    x = _lrelu(x + b1_ref[...])
    nf = _lrelu(jnp.dot(x, w2_ref[...], preferred_element_type=jnp.float32)
                + b2_ref[...])
    nf_ref[...] = nf
    ab_ref[:, :D] = jnp.dot(nf, wms_ref[...], preferred_element_type=jnp.float32)
    ab_ref[:, D:] = jnp.dot(nf, wmd_ref[...], preferred_element_type=jnp.float32)


@jax.jit
def _k1(pos, W1, b1, W2, b2, Wms, Wmd):
    return pl.pallas_call(
        _k1_body,
        out_shape=(jax.ShapeDtypeStruct((N_NODES, D), jnp.float32),
                   jax.ShapeDtypeStruct((N_NODES, 2 * D), jnp.float32)),
    )(pos, W1, b1.reshape(1, D), W2, b2.reshape(1, D), Wms, Wmd)


def _k2_body(ab_hbm, aag_hbm, bag_hbm, wme_hbm, bm_hbm, src_hbm, d64_hbm,
             ef0_hbm, ef1_hbm, ef2_hbm, mf_hbm, bwd_hbm,
             aag_v, bag_v, wme_v, bm_v, src_v, didx_v,
             ef0_v, ef1_v, ef2_v, rows_v, mfwd_v, mbwd_v, aggag_v, sem):
    c = lax.axis_index("c")
    s = lax.axis_index("s")
    wid = c * NS + s

    zrow = jnp.zeros((16,), jnp.float32)

    def zb(r, _):
        for k in range(4):
            aggag_v[r, pl.ds(16 * k, 16)] = zrow
        return 0
    lax.fori_loop(0, N_AG, zb, 0)

    pltpu.sync_copy(aag_hbm, aag_v)
    pltpu.sync_copy(bag_hbm, bag_v)
    pltpu.sync_copy(wme_hbm, wme_v)
    pltpu.sync_copy(bm_hbm, bm_v)

    w0 = [wme_v[0, pl.ds(16 * k, 16)] for k in range(4)]
    w1 = [wme_v[1, pl.ds(16 * k, 16)] for k in range(4)]
    w2 = [wme_v[2, pl.ds(16 * k, 16)] for k in range(4)]
    bmv = [bm_v[pl.ds(16 * k, 16)] for k in range(4)]

    def batch(b, _):
        eb = wid * EPT + b * BB
        bidx = wid * (EPT // BB) + b
        pltpu.sync_copy(src_hbm.at[bidx], src_v)
        pltpu.sync_copy(d64_hbm.at[bidx], didx_v)
        pltpu.sync_copy(ef0_hbm.at[pl.ds(eb, BB)], ef0_v)
        pltpu.sync_copy(ef1_hbm.at[pl.ds(eb, BB)], ef1_v)
        pltpu.sync_copy(ef2_hbm.at[pl.ds(eb, BB)], ef2_v)
        pltpu.async_copy(ab_hbm.at[didx_v.at[0]], rows_v, sem).wait()

        def grp(g, _):
            src16 = src_v[0, pl.ds(g * 16, 16)]
            e0_16 = ef0_v[pl.ds(g * 16, 16)]
            e1_16 = ef1_v[pl.ds(g * 16, 16)]
            e2_16 = ef2_v[pl.ds(g * 16, 16)]
            for lane in range(16):
                i = g * 16 + lane
                si = src16[lane]
                e0 = e0_16[lane]
                e1 = e1_16[lane]
                e2 = e2_16[lane]
                for k in range(4):
                    sl = pl.ds(16 * k, 16)
                    cvec = w0[k] * e0 + w1[k] * e1 + w2[k] * e2 + bmv[k]
                    ad = rows_v[i, sl]
                    bd = rows_v[i, pl.ds(D + 16 * k, 16)]
                    asv = aag_v[si, sl]
                    bsv = bag_v[si, sl]
                    mfwd_v[i, sl] = _lrelu(asv + bd + cvec)
                    aggag_v[si, sl] = aggag_v[si, sl] + _lrelu(ad + bsv + cvec)
            return 0
        lax.fori_loop(0, BB // 16, grp, 0)
        pltpu.sync_copy(mfwd_v, mf_hbm.at[pl.ds(eb, BB)])
        return 0
    lax.fori_loop(0, EPT // BB, batch, 0)

    pltpu.sync_copy(aggag_v, bwd_hbm.at[pl.ds(wid * N_AG, N_AG)])


@jax.jit
def _k2(ab, aag, bag, wme, bm, src, d64, ef0, ef1, ef2):
    mesh = plsc.VectorSubcoreMesh(core_axis_name="c", subcore_axis_name="s")
    f = pl.kernel(
        _k2_body,
        out_type=(jax.ShapeDtypeStruct((E, D), jnp.float32),
                  jax.ShapeDtypeStruct((NW * N_AG, D), jnp.float32)),
        mesh=mesh,
        scratch_types=[
            pltpu.VMEM((N_AG, D), jnp.float32),
            pltpu.VMEM((N_AG, D), jnp.float32),
            pltpu.VMEM((3, D), jnp.float32),
            pltpu.VMEM((D,), jnp.float32),
            pltpu.VMEM((1, BB), jnp.int32),
            pltpu.VMEM((1, BB), jnp.int32),
            pltpu.VMEM((BB,), jnp.float32),
            pltpu.VMEM((BB,), jnp.float32),
            pltpu.VMEM((BB,), jnp.float32),
            pltpu.VMEM((BB, 2 * D), jnp.float32),
            pltpu.VMEM((BB, D), jnp.float32),
            pltpu.VMEM((BB, D), jnp.float32),
            pltpu.VMEM((N_AG, D), jnp.float32),
            pltpu.SemaphoreType.DMA,
        ],
    )
    return f(ab, aag, bag, wme, bm,
             src.reshape(E // BB, 1, BB), d64.reshape(E // BB, 1, BB),
             ef0, ef1, ef2)


def _k3_body(aggp_ref, nf_ref, wu1_ref, wu2_ref, bu_ref, wb1s_ref, wb1d_ref,
             bb1_ref, pag_ref, q_ref):
    agg = aggp_ref[...]
    nf = nf_ref[...]
    u = (jnp.dot(nf, wu1_ref[...], preferred_element_type=jnp.float32)
         + jnp.dot(agg, wu2_ref[...], preferred_element_type=jnp.float32)
         + bu_ref[...])
    h = nf + _lrelu(u)
    q_ref[:, :D] = jnp.dot(h, wb1d_ref[...], preferred_element_type=jnp.float32)
    q_ref[:, D:] = jnp.zeros((N_NODES, D), jnp.float32)
    pag_ref[...] = (jnp.dot(h[:N_AG], wb1s_ref[...],
                            preferred_element_type=jnp.float32) + bb1_ref[...])


@jax.jit
def _k3(aggp, nf, Wu1, Wu2, bu, Wb1s, Wb1d, bb1):
    return pl.pallas_call(
        _k3_body,
        out_shape=(jax.ShapeDtypeStruct((N_AG, D), jnp.float32),
                   jax.ShapeDtypeStruct((N_NODES, 2 * D), jnp.float32)),
    )(aggp, nf, Wu1, Wu2, bu.reshape(1, D), Wb1s, Wb1d, bb1.reshape(1, D))


def _hsum16(acc, iota):
    for sh in (8, 4, 2, 1):
        p = (iota + sh) & 15
        acc = acc + acc.at[p].get(mode="promise_in_bounds")
    return acc[0]


def _bf16r(x):
    c = x * jnp.float32(65537.0)
    return c - (c - x)


def _k4_body(q_hbm, pag_hbm, wb2_hbm, src_hbm, d64_hbm, s_hbm,
             pag_v, wb2_v, src_v, d64_v, qrows_v, accs_v, sem):
    c = lax.axis_index("c")
    s = lax.axis_index("s")
    wid = c * NS + s

    pltpu.sync_copy(pag_hbm, pag_v)
    pltpu.sync_copy(wb2_hbm, wb2_v)
    wvecs = [wb2_v[pl.ds(16 * k, 16)] for k in range(4)]

    def batch(b, _):
        eb = wid * EPT + b * BB
        bidx = wid * (EPT // BB) + b
        pltpu.sync_copy(src_hbm.at[bidx], src_v)
        pltpu.sync_copy(d64_hbm.at[bidx], d64_v)
        pltpu.async_copy(q_hbm.at[d64_v.at[0]], qrows_v, sem).wait()

        def grp(g, _):
            src16 = src_v[0, pl.ds(g * 16, 16)]
            for lane in range(16):
                i = g * 16 + lane
                si = src16[lane]
                acc = jnp.zeros((16,), jnp.float32)
                for k in range(4):
                    sl = pl.ds(16 * k, 16)
                    t = _bf16r(_lrelu(pag_v[si, sl] + qrows_v[i, sl]))
                    acc = acc + t * wvecs[k]
                accs_v[i, pl.ds(0, 16)] = acc
            return 0
        lax.fori_loop(0, BB // 16, grp, 0)
        pltpu.sync_copy(accs_v, s_hbm.at[pl.ds(eb, BB)])
        return 0
    lax.fori_loop(0, EPT // BB, batch, 0)


@jax.jit
def _k4(q, pag, wb2, src, d64):
    mesh = plsc.VectorSubcoreMesh(core_axis_name="c", subcore_axis_name="s")
    f = pl.kernel(
        _k4_body,
        out_type=jax.ShapeDtypeStruct((E, 16), jnp.float32),
        mesh=mesh,
        scratch_types=[
            pltpu.VMEM((N_AG, D), jnp.float32),
            pltpu.VMEM((D,), jnp.float32),
            pltpu.VMEM((1, BB), jnp.int32),
            pltpu.VMEM((1, BB), jnp.int32),
            pltpu.VMEM((BB, 2 * D), jnp.float32),
            pltpu.VMEM((BB, 16), jnp.float32),
            pltpu.SemaphoreType.DMA,
        ],
    )
    return f(q, pag, wb2,
             src.reshape(E // BB, 1, BB), d64.reshape(E // BB, 1, BB))


def _k6_body(ao_ref, cont_ref, jap_ref, lg_ref, g_ref, out_ref,
             pol_ref, tk_ref, act_ref):
    lg = lg_ref[...]
    ex = jnp.exp(lg - jnp.max(lg, axis=1, keepdims=True))
    pol_ref[...] = ex / jnp.sum(ex, axis=1, keepdims=True)
    tk_ref[...] = jnp.zeros_like(tk_ref)
    act_ref[...] = jnp.zeros_like(act_ref)
    iota = lax.broadcasted_iota(jnp.int32, (1, N_TASK), 1)

    def body(itr, _):
        agent = ao_ref[itr]
        taken = tk_ref[...]
        row = jnp.where(taken != 0, 0.0, pol_ref[pl.ds(agent, 1), :])
        allzero = jnp.max(row) <= 0.0
        sc = jnp.log(jnp.clip(row, 1e-20, None)) + g_ref[pl.ds(itr, 1), :]
        m2 = jnp.max(sc)
        samp = jnp.min(jnp.where(sc == m2, iota, jnp.int32(2**31 - 1)))
        action = jnp.where(cont_ref[agent] != 0, jap_ref[agent], samp)
        action = jnp.where(allzero, jnp.int32(-1), action).astype(jnp.int32)
        lane = lax.broadcasted_iota(jnp.int32, (1, N_AG), 1)
        act_ref[...] = jnp.where(lane == itr, action, act_ref[...])
        col = jnp.maximum(action, 0)
        tk_ref[...] = jnp.where((iota == col) & jnp.logical_not(allzero),
                                jnp.int32(1), taken)
        return 0

    lax.fori_loop(0, N_AG, body, 0)
    out_ref[...] = act_ref[...]


@jax.jit
def _k6(logits, G, ag_order, cont_i32, jap):
    out = pl.pallas_call(
        _k6_body,
        out_shape=jax.ShapeDtypeStruct((1, N_AG), jnp.int32),
        in_specs=[
            pl.BlockSpec(memory_space=pltpu.SMEM),
            pl.BlockSpec(memory_space=pltpu.SMEM),
            pl.BlockSpec(memory_space=pltpu.SMEM),
            pl.BlockSpec(memory_space=pltpu.VMEM),
            pl.BlockSpec(memory_space=pltpu.VMEM),
        ],
        out_specs=pl.BlockSpec(memory_space=pltpu.VMEM),
        scratch_shapes=[
            pltpu.VMEM((N_AG, N_TASK), jnp.float32),
            pltpu.VMEM((1, N_TASK), jnp.int32),
            pltpu.VMEM((1, N_AG), jnp.int32),
        ],
    )(ag_order, cont_i32, jap, logits, G)
    return out[0]


def _gumbel_table():
    base = jax.random.key(42)
    subs = jax.vmap(lambda i: jax.random.fold_in(base, i))(jnp.arange(N_AG))
    return jax.vmap(lambda k: jax.random.gumbel(k, (N_TASK,), jnp.float32))(subs)


def kernel(pos, ef, edge_src, edge_dst, ag_order, continuing_ag, joint_action_prev,
           W1, b1, W2, b2, Wm, bm, Wu, bu, Wb1, bb1, Wb2, bb2):
    d64 = edge_dst + N_AG
    nf, ab = _k1(pos, W1, b1, W2, b2, Wm[:D], Wm[D:2 * D])
    aag = ab[:N_AG, :D]
    bag = ab[:N_AG, D:]
    efb = ef.astype(jnp.bfloat16).astype(jnp.float32)
    wmeb = Wm[2 * D:].astype(jnp.bfloat16).astype(jnp.float32)
    mf, bwd = _k2(ab, aag, bag, wmeb, bm, edge_src, d64,
                  efb[:, 0], efb[:, 1], efb[:, 2])
    agg = jnp.zeros((N_NODES, D), jnp.float32).at[d64].add(mf)
    agg = agg.at[:N_AG].add(bwd.reshape(NW, N_AG, D).sum(0))
    pag, q = _k3(agg, nf, Wu[:D], Wu[D:], bu, Wb1[:D], Wb1[D:], bb1)
    wb2b = Wb2[:, 0].astype(jnp.bfloat16).astype(jnp.float32)
    parts = _k4(q, pag, wb2b, edge_src, d64)
    scores = parts.sum(axis=1) + bb2[0]
    logits = jnp.full((N_AG, N_TASK), -1e9, jnp.float32).at[
        edge_src, edge_dst].max(scores)
    G = _gumbel_table()
    return _k6(logits, G, ag_order, continuing_ag.astype(jnp.int32),
               joint_action_prev.astype(jnp.int32))


# speedup vs baseline: 3.0723x; 1.0629x over previous
import jax
import jax.numpy as jnp
from jax import lax
from jax.experimental import pallas as pl
from jax.experimental.pallas import tpu as pltpu
from jax.experimental.pallas import tpu_sc as plsc

N_AG = 64
N_TASK = 8192
N_NODES = N_AG + N_TASK
E = 262144
D = 64

NC = 2
NS = 16
NW = NC * NS
EPT = E // NW
BB = 128


def _lrelu(x):
    return jnp.where(x >= 0, x, 0.01 * x)


def _k1_body(pos_ref, w1_ref, b1_ref, w2_ref, b2_ref, wms_ref, wmd_ref,
             nf_ref, ab_ref):
    x = jnp.dot(pos_ref[...], w1_ref[...], preferred_element_type=jnp.float32)
    x = _lrelu(x + b1_ref[...])
    nf = _lrelu(jnp.dot(x, w2_ref[...], preferred_element_type=jnp.float32)
                + b2_ref[...])
    nf_ref[...] = nf
    ab_ref[:, :D] = jnp.dot(nf, wms_ref[...], preferred_element_type=jnp.float32)
    ab_ref[:, D:] = jnp.dot(nf, wmd_ref[...], preferred_element_type=jnp.float32)


@jax.jit
def _k1(pos, W1, b1, W2, b2, Wms, Wmd):
    return pl.pallas_call(
        _k1_body,
        out_shape=(jax.ShapeDtypeStruct((N_NODES, D), jnp.float32),
                   jax.ShapeDtypeStruct((N_NODES, 2 * D), jnp.float32)),
    )(pos, W1, b1.reshape(1, D), W2, b2.reshape(1, D), Wms, Wmd)


def _k2_body(ab_hbm, aag_hbm, bag_hbm, wme_hbm, bm_hbm, src_hbm, d64_hbm,
             ef0_hbm, ef1_hbm, ef2_hbm, mf_hbm, bwd_hbm,
             aag_v, bag_v, wme_v, bm_v, src_v, didx_v,
             ef0_v, ef1_v, ef2_v, rows_v, mfwd_v, mbwd_v, aggag_v, sem, sem2):
    c = lax.axis_index("c")
    s = lax.axis_index("s")
    wid = c * NS + s

    zrow = jnp.zeros((16,), jnp.float32)

    def zb(r, _):
        for k in range(4):
            aggag_v[r, pl.ds(16 * k, 16)] = zrow
        return 0
    lax.fori_loop(0, N_AG, zb, 0)

    pltpu.sync_copy(aag_hbm, aag_v)
    pltpu.sync_copy(bag_hbm, bag_v)
    pltpu.sync_copy(wme_hbm, wme_v)
    pltpu.sync_copy(bm_hbm, bm_v)

    w0 = [wme_v[0, pl.ds(16 * k, 16)] for k in range(4)]
    w1 = [wme_v[1, pl.ds(16 * k, 16)] for k in range(4)]
    w2 = [wme_v[2, pl.ds(16 * k, 16)] for k in range(4)]
    bmv = [bm_v[pl.ds(16 * k, 16)] for k in range(4)]

    def batch(b, _):
        eb = wid * EPT + b * BB
        bidx = wid * (EPT // BB) + b
        pltpu.sync_copy(d64_hbm.at[bidx], didx_v)
        cg = pltpu.async_copy(ab_hbm.at[didx_v.at[0]], rows_v, sem)
        c1 = pltpu.async_copy(src_hbm.at[bidx], src_v, sem2)
        c2 = pltpu.async_copy(ef0_hbm.at[pl.ds(eb, BB)], ef0_v, sem2)
        c3 = pltpu.async_copy(ef1_hbm.at[pl.ds(eb, BB)], ef1_v, sem2)
        c4 = pltpu.async_copy(ef2_hbm.at[pl.ds(eb, BB)], ef2_v, sem2)
        c1.wait()
        c2.wait()
        c3.wait()
        c4.wait()
        cg.wait()

        def grp(g, _):
            src16 = src_v[0, pl.ds(g * 16, 16)]
            e0_16 = ef0_v[pl.ds(g * 16, 16)]
            e1_16 = ef1_v[pl.ds(g * 16, 16)]
            e2_16 = ef2_v[pl.ds(g * 16, 16)]
            for lane in range(16):
                i = g * 16 + lane
                si = src16[lane]
                e0 = e0_16[lane]
                e1 = e1_16[lane]
                e2 = e2_16[lane]
                for k in range(4):
                    sl = pl.ds(16 * k, 16)
                    cvec = w0[k] * e0 + w1[k] * e1 + w2[k] * e2 + bmv[k]
                    ad = rows_v[i, sl]
                    bd = rows_v[i, pl.ds(D + 16 * k, 16)]
                    asv = aag_v[si, sl]
                    bsv = bag_v[si, sl]
                    mfwd_v[i, sl] = _lrelu(asv + bd + cvec)
                    aggag_v[si, sl] = aggag_v[si, sl] + _lrelu(ad + bsv + cvec)
            return 0
        lax.fori_loop(0, BB // 16, grp, 0)
        pltpu.sync_copy(mfwd_v, mf_hbm.at[pl.ds(eb, BB)])
        return 0
    lax.fori_loop(0, EPT // BB, batch, 0)

    pltpu.sync_copy(aggag_v, bwd_hbm.at[pl.ds(wid * N_AG, N_AG)])


@jax.jit
def _k2(ab, aag, bag, wme, bm, src, d64, ef0, ef1, ef2):
    mesh = plsc.VectorSubcoreMesh(core_axis_name="c", subcore_axis_name="s")
    f = pl.kernel(
        _k2_body,
        out_type=(jax.ShapeDtypeStruct((E, D), jnp.float32),
                  jax.ShapeDtypeStruct((NW * N_AG, D), jnp.float32)),
        mesh=mesh,
        scratch_types=[
            pltpu.VMEM((N_AG, D), jnp.float32),
            pltpu.VMEM((N_AG, D), jnp.float32),
            pltpu.VMEM((3, D), jnp.float32),
            pltpu.VMEM((D,), jnp.float32),
            pltpu.VMEM((1, BB), jnp.int32),
            pltpu.VMEM((1, BB), jnp.int32),
            pltpu.VMEM((BB,), jnp.float32),
            pltpu.VMEM((BB,), jnp.float32),
            pltpu.VMEM((BB,), jnp.float32),
            pltpu.VMEM((BB, 2 * D), jnp.float32),
            pltpu.VMEM((BB, D), jnp.float32),
            pltpu.VMEM((BB, D), jnp.float32),
            pltpu.VMEM((N_AG, D), jnp.float32),
            pltpu.SemaphoreType.DMA,
            pltpu.SemaphoreType.DMA,
        ],
    )
    return f(ab, aag, bag, wme, bm,
             src.reshape(E // BB, 1, BB), d64.reshape(E // BB, 1, BB),
             ef0, ef1, ef2)


def _k3_body(aggp_ref, nf_ref, wu1_ref, wu2_ref, bu_ref, wb1s_ref, wb1d_ref,
             bb1_ref, pag_ref, q_ref):
    agg = aggp_ref[...]
    nf = nf_ref[...]
    u = (jnp.dot(nf, wu1_ref[...], preferred_element_type=jnp.float32)
         + jnp.dot(agg, wu2_ref[...], preferred_element_type=jnp.float32)
         + bu_ref[...])
    h = nf + _lrelu(u)
    q_ref[:, :D] = jnp.dot(h, wb1d_ref[...], preferred_element_type=jnp.float32)
    q_ref[:, D:] = jnp.zeros((N_NODES, D), jnp.float32)
    pag_ref[...] = (jnp.dot(h[:N_AG], wb1s_ref[...],
                            preferred_element_type=jnp.float32) + bb1_ref[...])


@jax.jit
def _k3(aggp, nf, Wu1, Wu2, bu, Wb1s, Wb1d, bb1):
    return pl.pallas_call(
        _k3_body,
        out_shape=(jax.ShapeDtypeStruct((N_AG, D), jnp.float32),
                   jax.ShapeDtypeStruct((N_NODES, 2 * D), jnp.float32)),
    )(aggp, nf, Wu1, Wu2, bu.reshape(1, D), Wb1s, Wb1d, bb1.reshape(1, D))


def _hsum16(acc, iota):
    for sh in (8, 4, 2, 1):
        p = (iota + sh) & 15
        acc = acc + acc.at[p].get(mode="promise_in_bounds")
    return acc[0]


def _bf16r(x):
    c = x * jnp.float32(65537.0)
    return c - (c - x)


def _k4_body(q_hbm, pag_hbm, wb2_hbm, src_hbm, d64_hbm, s_hbm,
             pag_v, wb2_v, src_v, d64_v, qrows_v, accs_v, sem, sem2):
    c = lax.axis_index("c")
    s = lax.axis_index("s")
    wid = c * NS + s

    pltpu.sync_copy(pag_hbm, pag_v)
    pltpu.sync_copy(wb2_hbm, wb2_v)
    wvecs = [wb2_v[pl.ds(16 * k, 16)] for k in range(4)]

    def batch(b, _):
        eb = wid * EPT + b * BB
        bidx = wid * (EPT // BB) + b
        pltpu.sync_copy(d64_hbm.at[bidx], d64_v)
        cg = pltpu.async_copy(q_hbm.at[d64_v.at[0]], qrows_v, sem)
        c1 = pltpu.async_copy(src_hbm.at[bidx], src_v, sem2)
        c1.wait()
        cg.wait()

        def grp(g, _):
            src16 = src_v[0, pl.ds(g * 16, 16)]
            for lane in range(16):
                i = g * 16 + lane
                si = src16[lane]
                acc = jnp.zeros((16,), jnp.float32)
                for k in range(4):
                    sl = pl.ds(16 * k, 16)
                    t = _bf16r(_lrelu(pag_v[si, sl] + qrows_v[i, sl]))
                    acc = acc + t * wvecs[k]
                accs_v[i, pl.ds(0, 16)] = acc
            return 0
        lax.fori_loop(0, BB // 16, grp, 0)
        pltpu.sync_copy(accs_v, s_hbm.at[pl.ds(eb, BB)])
        return 0
    lax.fori_loop(0, EPT // BB, batch, 0)


@jax.jit
def _k4(q, pag, wb2, src, d64):
    mesh = plsc.VectorSubcoreMesh(core_axis_name="c", subcore_axis_name="s")
    f = pl.kernel(
        _k4_body,
        out_type=jax.ShapeDtypeStruct((E, 16), jnp.float32),
        mesh=mesh,
        scratch_types=[
            pltpu.VMEM((N_AG, D), jnp.float32),
            pltpu.VMEM((D,), jnp.float32),
            pltpu.VMEM((1, BB), jnp.int32),
            pltpu.VMEM((1, BB), jnp.int32),
            pltpu.VMEM((BB, 2 * D), jnp.float32),
            pltpu.VMEM((BB, 16), jnp.float32),
            pltpu.SemaphoreType.DMA,
            pltpu.SemaphoreType.DMA,
        ],
    )
    return f(q, pag, wb2,
             src.reshape(E // BB, 1, BB), d64.reshape(E // BB, 1, BB))


def _k6_body(ao_ref, cont_ref, jap_ref, lg_ref, g_ref, out_ref,
             pol_ref, tk_ref, act_ref):
    lg = lg_ref[...]
    ex = jnp.exp(lg - jnp.max(lg, axis=1, keepdims=True))
    pol_ref[...] = ex / jnp.sum(ex, axis=1, keepdims=True)
    tk_ref[...] = jnp.zeros_like(tk_ref)
    act_ref[...] = jnp.zeros_like(act_ref)
    iota = lax.broadcasted_iota(jnp.int32, (1, N_TASK), 1)

    def body(itr, _):
        agent = ao_ref[itr]
        taken = tk_ref[...]
        row = jnp.where(taken != 0, 0.0, pol_ref[pl.ds(agent, 1), :])
        allzero = jnp.max(row) <= 0.0
        sc = jnp.log(jnp.clip(row, 1e-20, None)) + g_ref[pl.ds(itr, 1), :]
        m2 = jnp.max(sc)
        samp = jnp.min(jnp.where(sc == m2, iota, jnp.int32(2**31 - 1)))
        action = jnp.where(cont_ref[agent] != 0, jap_ref[agent], samp)
        action = jnp.where(allzero, jnp.int32(-1), action).astype(jnp.int32)
        lane = lax.broadcasted_iota(jnp.int32, (1, N_AG), 1)
        act_ref[...] = jnp.where(lane == itr, action, act_ref[...])
        col = jnp.maximum(action, 0)
        tk_ref[...] = jnp.where((iota == col) & jnp.logical_not(allzero),
                                jnp.int32(1), taken)
        return 0

    lax.fori_loop(0, N_AG, body, 0)
    out_ref[...] = act_ref[...]


@jax.jit
def _k6(logits, G, ag_order, cont_i32, jap):
    out = pl.pallas_call(
        _k6_body,
        out_shape=jax.ShapeDtypeStruct((1, N_AG), jnp.int32),
        in_specs=[
            pl.BlockSpec(memory_space=pltpu.SMEM),
            pl.BlockSpec(memory_space=pltpu.SMEM),
            pl.BlockSpec(memory_space=pltpu.SMEM),
            pl.BlockSpec(memory_space=pltpu.VMEM),
            pl.BlockSpec(memory_space=pltpu.VMEM),
        ],
        out_specs=pl.BlockSpec(memory_space=pltpu.VMEM),
        scratch_shapes=[
            pltpu.VMEM((N_AG, N_TASK), jnp.float32),
            pltpu.VMEM((1, N_TASK), jnp.int32),
            pltpu.VMEM((1, N_AG), jnp.int32),
        ],
    )(ag_order, cont_i32, jap, logits, G)
    return out[0]


def _gumbel_table():
    base = jax.random.key(42)
    subs = jax.vmap(lambda i: jax.random.fold_in(base, i))(jnp.arange(N_AG))
    return jax.vmap(lambda k: jax.random.gumbel(k, (N_TASK,), jnp.float32))(subs)


def kernel(pos, ef, edge_src, edge_dst, ag_order, continuing_ag, joint_action_prev,
           W1, b1, W2, b2, Wm, bm, Wu, bu, Wb1, bb1, Wb2, bb2):
    d64 = edge_dst + N_AG
    nf, ab = _k1(pos, W1, b1, W2, b2, Wm[:D], Wm[D:2 * D])
    aag = ab[:N_AG, :D]
    bag = ab[:N_AG, D:]
    efb = ef.astype(jnp.bfloat16).astype(jnp.float32)
    wmeb = Wm[2 * D:].astype(jnp.bfloat16).astype(jnp.float32)
    mf, bwd = _k2(ab, aag, bag, wmeb, bm, edge_src, d64,
                  efb[:, 0], efb[:, 1], efb[:, 2])
    agg = jnp.zeros((N_NODES, D), jnp.float32).at[d64].add(mf)
    agg = agg.at[:N_AG].add(bwd.reshape(NW, N_AG, D).sum(0))
    pag, q = _k3(agg, nf, Wu[:D], Wu[D:], bu, Wb1[:D], Wb1[D:], bb1)
    wb2b = Wb2[:, 0].astype(jnp.bfloat16).astype(jnp.float32)
    parts = _k4(q, pag, wb2b, edge_src, d64)
    scores = parts.sum(axis=1) + bb2[0]
    logits = jnp.full((N_AG, N_TASK), -1e9, jnp.float32).at[
        edge_src, edge_dst].max(scores)
    G = _gumbel_table()
    return _k6(logits, G, ag_order, continuing_ag.astype(jnp.int32),
               joint_action_prev.astype(jnp.int32))


# speedup vs baseline: 3.0876x; 1.0050x over previous
import jax
import jax.numpy as jnp
from jax import lax
from jax.experimental import pallas as pl
from jax.experimental.pallas import tpu as pltpu
from jax.experimental.pallas import tpu_sc as plsc

N_AG = 64
N_TASK = 8192
N_NODES = N_AG + N_TASK
E = 262144
D = 64

NC = 2
NS = 16
NW = NC * NS
EPT = E // NW
BB = 128


def _lrelu(x):
    return jnp.where(x >= 0, x, 0.01 * x)


def _k1_body(pos_ref, w1_ref, b1_ref, w2_ref, b2_ref, wms_ref, wmd_ref,
             nf_ref, ab_ref):
    x = jnp.dot(pos_ref[...], w1_ref[...], preferred_element_type=jnp.float32)
    x = _lrelu(x + b1_ref[...])
    nf = _lrelu(jnp.dot(x, w2_ref[...], preferred_element_type=jnp.float32)
                + b2_ref[...])
    nf_ref[...] = nf
    ab_ref[:, :D] = jnp.dot(nf, wms_ref[...], preferred_element_type=jnp.float32)
    ab_ref[:, D:] = jnp.dot(nf, wmd_ref[...], preferred_element_type=jnp.float32)


@jax.jit
def _k1(pos, W1, b1, W2, b2, Wms, Wmd):
    return pl.pallas_call(
        _k1_body,
        out_shape=(jax.ShapeDtypeStruct((N_NODES, D), jnp.float32),
                   jax.ShapeDtypeStruct((N_NODES, 2 * D), jnp.float32)),
    )(pos, W1, b1.reshape(1, D), W2, b2.reshape(1, D), Wms, Wmd)


def _k2_body(ab_hbm, aag_hbm, bag_hbm, wme_hbm, bm_hbm, src_hbm, d64_hbm,
             ef0_hbm, ef1_hbm, ef2_hbm, mf_hbm, bwd_hbm,
             aag_v, bag_v, wme_v, bm_v, src_v, didx_v,
             ef0_v, ef1_v, ef2_v, rows_v, mfwd_v, mbwd_v, aggag_v, sem, sem2):
    c = lax.axis_index("c")
    s = lax.axis_index("s")
    wid = c * NS + s

    zrow = jnp.zeros((16,), jnp.float32)

    def zb(r, _):
        for k in range(4):
            aggag_v[r, pl.ds(16 * k, 16)] = zrow
        return 0
    lax.fori_loop(0, N_AG, zb, 0)

    pltpu.sync_copy(aag_hbm, aag_v)
    pltpu.sync_copy(bag_hbm, bag_v)
    pltpu.sync_copy(wme_hbm, wme_v)
    pltpu.sync_copy(bm_hbm, bm_v)

    w0 = [wme_v[0, pl.ds(16 * k, 16)] for k in range(4)]
    w1 = [wme_v[1, pl.ds(16 * k, 16)] for k in range(4)]
    w2 = [wme_v[2, pl.ds(16 * k, 16)] for k in range(4)]
    bmv = [bm_v[pl.ds(16 * k, 16)] for k in range(4)]

    def batch(b, _):
        eb = wid * EPT + b * BB
        bidx = wid * (EPT // BB) + b
        sb = lax.rem(b, 4)
        base_i = sb * BB
        pltpu.sync_copy(d64_hbm.at[bidx], didx_v)
        cg = pltpu.async_copy(ab_hbm.at[didx_v.at[0]], rows_v, sem)

        @pl.when(sb == 0)
        def _():
            bidx2 = wid * (EPT // (4 * BB)) + lax.div(b, 4)
            c1 = pltpu.async_copy(src_hbm.at[bidx2], src_v, sem2)
            c2 = pltpu.async_copy(ef0_hbm.at[pl.ds(eb, 4 * BB)], ef0_v, sem2)
            c3 = pltpu.async_copy(ef1_hbm.at[pl.ds(eb, 4 * BB)], ef1_v, sem2)
            c4 = pltpu.async_copy(ef2_hbm.at[pl.ds(eb, 4 * BB)], ef2_v, sem2)
            c1.wait()
            c2.wait()
            c3.wait()
            c4.wait()
        cg.wait()

        def grp(g, _):
            src16 = src_v[0, pl.ds(base_i + g * 16, 16)]
            e0_16 = ef0_v[pl.ds(base_i + g * 16, 16)]
            e1_16 = ef1_v[pl.ds(base_i + g * 16, 16)]
            e2_16 = ef2_v[pl.ds(base_i + g * 16, 16)]
            for lane in range(16):
                i = g * 16 + lane
                si = src16[lane]
                e0 = e0_16[lane]
                e1 = e1_16[lane]
                e2 = e2_16[lane]
                for k in range(4):
                    sl = pl.ds(16 * k, 16)
                    cvec = w0[k] * e0 + w1[k] * e1 + w2[k] * e2 + bmv[k]
                    ad = rows_v[i, sl]
                    bd = rows_v[i, pl.ds(D + 16 * k, 16)]
                    asv = aag_v[si, sl]
                    bsv = bag_v[si, sl]
                    mfwd_v[base_i + i, sl] = _lrelu(asv + bd + cvec)
                    aggag_v[si, sl] = aggag_v[si, sl] + _lrelu(ad + bsv + cvec)
            return 0
        lax.fori_loop(0, BB // 16, grp, 0)

        @pl.when(sb == 3)
        def _():
            pltpu.sync_copy(mfwd_v, mf_hbm.at[pl.ds(eb - 3 * BB, 4 * BB)])
        return 0
    lax.fori_loop(0, EPT // BB, batch, 0)

    pltpu.sync_copy(aggag_v, bwd_hbm.at[pl.ds(wid * N_AG, N_AG)])


@jax.jit
def _k2(ab, aag, bag, wme, bm, src, d64, ef0, ef1, ef2):
    mesh = plsc.VectorSubcoreMesh(core_axis_name="c", subcore_axis_name="s")
    f = pl.kernel(
        _k2_body,
        out_type=(jax.ShapeDtypeStruct((E, D), jnp.float32),
                  jax.ShapeDtypeStruct((NW * N_AG, D), jnp.float32)),
        mesh=mesh,
        scratch_types=[
            pltpu.VMEM((N_AG, D), jnp.float32),
            pltpu.VMEM((N_AG, D), jnp.float32),
            pltpu.VMEM((3, D), jnp.float32),
            pltpu.VMEM((D,), jnp.float32),
            pltpu.VMEM((1, 4 * BB), jnp.int32),
            pltpu.VMEM((1, BB), jnp.int32),
            pltpu.VMEM((4 * BB,), jnp.float32),
            pltpu.VMEM((4 * BB,), jnp.float32),
            pltpu.VMEM((4 * BB,), jnp.float32),
            pltpu.VMEM((BB, 2 * D), jnp.float32),
            pltpu.VMEM((4 * BB, D), jnp.float32),
            pltpu.VMEM((BB, D), jnp.float32),
            pltpu.VMEM((N_AG, D), jnp.float32),
            pltpu.SemaphoreType.DMA,
            pltpu.SemaphoreType.DMA,
        ],
    )
    return f(ab, aag, bag, wme, bm,
             src.reshape(E // (4 * BB), 1, 4 * BB),
             d64.reshape(E // BB, 1, BB),
             ef0, ef1, ef2)


def _k3_body(aggp_ref, nf_ref, wu1_ref, wu2_ref, bu_ref, wb1s_ref, wb1d_ref,
             bb1_ref, pag_ref, q_ref):
    agg = aggp_ref[...]
    nf = nf_ref[...]
    u = (jnp.dot(nf, wu1_ref[...], preferred_element_type=jnp.float32)
         + jnp.dot(agg, wu2_ref[...], preferred_element_type=jnp.float32)
         + bu_ref[...])
    h = nf + _lrelu(u)
    q_ref[:, :D] = jnp.dot(h, wb1d_ref[...], preferred_element_type=jnp.float32)
    q_ref[:, D:] = jnp.zeros((N_NODES, D), jnp.float32)
    pag_ref[...] = (jnp.dot(h[:N_AG], wb1s_ref[...],
                            preferred_element_type=jnp.float32) + bb1_ref[...])


@jax.jit
def _k3(aggp, nf, Wu1, Wu2, bu, Wb1s, Wb1d, bb1):
    return pl.pallas_call(
        _k3_body,
        out_shape=(jax.ShapeDtypeStruct((N_AG, D), jnp.float32),
                   jax.ShapeDtypeStruct((N_NODES, 2 * D), jnp.float32)),
    )(aggp, nf, Wu1, Wu2, bu.reshape(1, D), Wb1s, Wb1d, bb1.reshape(1, D))


def _hsum16(acc, iota):
    for sh in (8, 4, 2, 1):
        p = (iota + sh) & 15
        acc = acc + acc.at[p].get(mode="promise_in_bounds")
    return acc[0]


def _bf16r(x):
    c = x * jnp.float32(65537.0)
    return c - (c - x)


def _k4_body(q_hbm, pag_hbm, wb2_hbm, src_hbm, d64_hbm, s_hbm,
             pag_v, wb2_v, src_v, d64_v, qrows_v, accs_v, sem, sem2):
    c = lax.axis_index("c")
    s = lax.axis_index("s")
    wid = c * NS + s

    pltpu.sync_copy(pag_hbm, pag_v)
    pltpu.sync_copy(wb2_hbm, wb2_v)
    wvecs = [wb2_v[pl.ds(16 * k, 16)] for k in range(4)]

    def batch(b, _):
        eb = wid * EPT + b * BB
        bidx = wid * (EPT // BB) + b
        pltpu.sync_copy(d64_hbm.at[bidx], d64_v)
        cg = pltpu.async_copy(q_hbm.at[d64_v.at[0]], qrows_v, sem)
        c1 = pltpu.async_copy(src_hbm.at[bidx], src_v, sem2)
        c1.wait()
        cg.wait()

        def grp(g, _):
            src16 = src_v[0, pl.ds(g * 16, 16)]
            for lane in range(16):
                i = g * 16 + lane
                si = src16[lane]
                acc = jnp.zeros((16,), jnp.float32)
                for k in range(4):
                    sl = pl.ds(16 * k, 16)
                    t = _bf16r(_lrelu(pag_v[si, sl] + qrows_v[i, sl]))
                    acc = acc + t * wvecs[k]
                accs_v[i, pl.ds(0, 16)] = acc
            return 0
        lax.fori_loop(0, BB // 16, grp, 0)
        pltpu.sync_copy(accs_v, s_hbm.at[pl.ds(eb, BB)])
        return 0
    lax.fori_loop(0, EPT // BB, batch, 0)


@jax.jit
def _k4(q, pag, wb2, src, d64):
    mesh = plsc.VectorSubcoreMesh(core_axis_name="c", subcore_axis_name="s")
    f = pl.kernel(
        _k4_body,
        out_type=jax.ShapeDtypeStruct((E, 16), jnp.float32),
        mesh=mesh,
        scratch_types=[
            pltpu.VMEM((N_AG, D), jnp.float32),
            pltpu.VMEM((D,), jnp.float32),
            pltpu.VMEM((1, BB), jnp.int32),
            pltpu.VMEM((1, BB), jnp.int32),
            pltpu.VMEM((BB, 2 * D), jnp.float32),
            pltpu.VMEM((BB, 16), jnp.float32),
            pltpu.SemaphoreType.DMA,
            pltpu.SemaphoreType.DMA,
        ],
    )
    return f(q, pag, wb2,
             src.reshape(E // BB, 1, BB), d64.reshape(E // BB, 1, BB))


def _k6_body(ao_ref, cont_ref, jap_ref, lg_ref, g_ref, out_ref,
             pol_ref, tk_ref, act_ref):
    lg = lg_ref[...]
    ex = jnp.exp(lg - jnp.max(lg, axis=1, keepdims=True))
    pol_ref[...] = ex / jnp.sum(ex, axis=1, keepdims=True)
    tk_ref[...] = jnp.zeros_like(tk_ref)
    act_ref[...] = jnp.zeros_like(act_ref)
    iota = lax.broadcasted_iota(jnp.int32, (1, N_TASK), 1)

    def body(itr, _):
        agent = ao_ref[itr]
        taken = tk_ref[...]
        row = jnp.where(taken != 0, 0.0, pol_ref[pl.ds(agent, 1), :])
        allzero = jnp.max(row) <= 0.0
        sc = jnp.log(jnp.clip(row, 1e-20, None)) + g_ref[pl.ds(itr, 1), :]
        m2 = jnp.max(sc)
        samp = jnp.min(jnp.where(sc == m2, iota, jnp.int32(2**31 - 1)))
        action = jnp.where(cont_ref[agent] != 0, jap_ref[agent], samp)
        action = jnp.where(allzero, jnp.int32(-1), action).astype(jnp.int32)
        lane = lax.broadcasted_iota(jnp.int32, (1, N_AG), 1)
        act_ref[...] = jnp.where(lane == itr, action, act_ref[...])
        col = jnp.maximum(action, 0)
        tk_ref[...] = jnp.where((iota == col) & jnp.logical_not(allzero),
                                jnp.int32(1), taken)
        return 0

    lax.fori_loop(0, N_AG, body, 0)
    out_ref[...] = act_ref[...]


@jax.jit
def _k6(logits, G, ag_order, cont_i32, jap):
    out = pl.pallas_call(
        _k6_body,
        out_shape=jax.ShapeDtypeStruct((1, N_AG), jnp.int32),
        in_specs=[
            pl.BlockSpec(memory_space=pltpu.SMEM),
            pl.BlockSpec(memory_space=pltpu.SMEM),
            pl.BlockSpec(memory_space=pltpu.SMEM),
            pl.BlockSpec(memory_space=pltpu.VMEM),
            pl.BlockSpec(memory_space=pltpu.VMEM),
        ],
        out_specs=pl.BlockSpec(memory_space=pltpu.VMEM),
        scratch_shapes=[
            pltpu.VMEM((N_AG, N_TASK), jnp.float32),
            pltpu.VMEM((1, N_TASK), jnp.int32),
            pltpu.VMEM((1, N_AG), jnp.int32),
        ],
    )(ag_order, cont_i32, jap, logits, G)
    return out[0]


def _gumbel_table():
    base = jax.random.key(42)
    subs = jax.vmap(lambda i: jax.random.fold_in(base, i))(jnp.arange(N_AG))
    return jax.vmap(lambda k: jax.random.gumbel(k, (N_TASK,), jnp.float32))(subs)


def kernel(pos, ef, edge_src, edge_dst, ag_order, continuing_ag, joint_action_prev,
           W1, b1, W2, b2, Wm, bm, Wu, bu, Wb1, bb1, Wb2, bb2):
    d64 = edge_dst + N_AG
    nf, ab = _k1(pos, W1, b1, W2, b2, Wm[:D], Wm[D:2 * D])
    aag = ab[:N_AG, :D]
    bag = ab[:N_AG, D:]
    efb = ef.astype(jnp.bfloat16).astype(jnp.float32)
    wmeb = Wm[2 * D:].astype(jnp.bfloat16).astype(jnp.float32)
    mf, bwd = _k2(ab, aag, bag, wmeb, bm, edge_src, d64,
                  efb[:, 0], efb[:, 1], efb[:, 2])
    agg = jnp.zeros((N_NODES, D), jnp.float32).at[d64].add(mf)
    agg = agg.at[:N_AG].add(bwd.reshape(NW, N_AG, D).sum(0))
    pag, q = _k3(agg, nf, Wu[:D], Wu[D:], bu, Wb1[:D], Wb1[D:], bb1)
    wb2b = Wb2[:, 0].astype(jnp.bfloat16).astype(jnp.float32)
    parts = _k4(q, pag, wb2b, edge_src, d64)
    scores = parts.sum(axis=1) + bb2[0]
    logits = jnp.full((N_AG, N_TASK), -1e9, jnp.float32).at[
        edge_src, edge_dst].max(scores)
    G = _gumbel_table()
    return _k6(logits, G, ag_order, continuing_ag.astype(jnp.int32),
               joint_action_prev.astype(jnp.int32))
